# baseline (speedup 1.0000x reference)
import functools

import jax
import jax.numpy as jnp
from jax import lax
from jax.experimental import pallas as pl
from jax.experimental.pallas import tpu as pltpu
from jax.experimental.pallas import tpu_sc as plsc

B = 16384
F = 64
NC = 2
NS = 16
NW = NC * NS
BPW = B // NW
L = 16
CH = 16
NCH = BPW // CH


def _body(uid, iid, text2, utab2, itab2, ub2, ib2, out,
          uidx_v, iidx_v, ubuf, ibuf, tbuf, ubbuf, ibbuf, outc, sem):
    wid = lax.axis_index("s") * NC + lax.axis_index("c")
    base = wid * BPW

    pltpu.sync_copy(uid.at[pl.ds(base, BPW)], uidx_v)
    pltpu.sync_copy(iid.at[pl.ds(base, BPW)], iidx_v)

    jv = lax.iota(jnp.int32, L)
    zv = jnp.zeros((L,), jnp.int32)

    def chunk(c, _):
        cb = c * CH
        uvec = uidx_v[pl.ds(cb, L)]
        ivec = iidx_v[pl.ds(cb, L)]
        usubv = lax.bitwise_and(uvec, 7)
        isubv = lax.bitwise_and(ivec, 7)
        ubasev = uvec - usubv
        ibasev = ivec - isubv

        copies = [pltpu.async_copy(
            text2.at[pl.ds(base + cb, CH)], tbuf, sem)]
        for j in range(CH):
            ub_ = pl.multiple_of(ubasev[j], 8)
            ib_ = pl.multiple_of(ibasev[j], 8)
            copies.append(pltpu.async_copy(
                utab2.at[pl.ds(ub_, 8)], ubuf.at[pl.ds(j * 8, 8)], sem))
            copies.append(pltpu.async_copy(
                itab2.at[pl.ds(ib_, 8)], ibuf.at[pl.ds(j * 8, 8)], sem))
            copies.append(pltpu.async_copy(
                ub2.at[pl.ds(ub_, 8)], ubbuf.at[pl.ds(j * 8, 8)], sem))
            copies.append(pltpu.async_copy(
                ib2.at[pl.ds(ib_, 8)], ibbuf.at[pl.ds(j * 8, 8)], sem))
        for cp in copies:
            cp.wait()

        rowu = jv * 8 + usubv
        rowi = jv * 8 + isubv
        acc = (plsc.load_gather(ubbuf, [rowu, zv])
               + plsc.load_gather(ibbuf, [rowi, zv]))
        for f in range(F):
            fv = jnp.full((L,), f, jnp.int32)
            u = plsc.load_gather(ubuf, [rowu, fv])
            i = plsc.load_gather(ibuf, [rowi, fv])
            t = plsc.load_gather(tbuf, [jv, fv])
            acc = acc + u * (i + t)
        plsc.store_scatter(outc, [jv, zv], acc)
        pltpu.sync_copy(outc, out.at[pl.ds(base + cb, CH)])
        return 0

    lax.fori_loop(0, NCH, chunk, 0)


def kernel(user_id, item_id, text_embeddings, user_table, item_table,
           user_bias, item_bias):
    mesh = plsc.VectorSubcoreMesh(core_axis_name="c", subcore_axis_name="s")
    k = functools.partial(
        pl.kernel,
        out_type=jax.ShapeDtypeStruct((B, 1), jnp.float32),
        mesh=mesh,
        compiler_params=pltpu.CompilerParams(
            needs_layout_passes=False, use_tc_tiling_on_sc=True),
        scratch_types=[
            pltpu.VMEM((BPW,), jnp.int32),
            pltpu.VMEM((BPW,), jnp.int32),
            pltpu.VMEM((CH * 8, F), jnp.float32),
            pltpu.VMEM((CH * 8, F), jnp.float32),
            pltpu.VMEM((CH, F), jnp.float32),
            pltpu.VMEM((CH * 8, 1), jnp.float32),
            pltpu.VMEM((CH * 8, 1), jnp.float32),
            pltpu.VMEM((CH, 1), jnp.float32),
            pltpu.SemaphoreType.DMA,
        ],
    )(_body)
    return k(user_id.reshape(B), item_id.reshape(B), text_embeddings,
             user_table, item_table, user_bias, item_bias)


# speedup vs baseline: 1.0975x; 1.0975x over previous
import functools

import jax
import jax.numpy as jnp
from jax import lax
from jax.experimental import pallas as pl
from jax.experimental.pallas import tpu as pltpu
from jax.experimental.pallas import tpu_sc as plsc

B = 16384
F = 64
NC = 2
NS = 16
NW = NC * NS
BPW = B // NW
L = 16
CH = 16
NCH = BPW // CH


def _body(uidT, iidT, textT, utab, itab, ubp, ibp, out2,
          uidx_v, iidx_v, ubb_v, ibb_v, ubuf, ibuf, tbufT, ubbuf, ibbuf,
          out_v, sem):
    wid = lax.axis_index("s") * NC + lax.axis_index("c")
    base = wid * BPW

    pltpu.sync_copy(uidT.at[0, pl.ds(base, BPW)], uidx_v)
    pltpu.sync_copy(iidT.at[0, pl.ds(base, BPW)], iidx_v)
    tcp = pltpu.async_copy(textT.at[:, pl.ds(base, BPW)], tbufT, sem)

    def prep(j, _):
        u = uidx_v[pl.ds(j * L, L)]
        i = iidx_v[pl.ds(j * L, L)]
        ubb_v[pl.ds(j * L, L)] = lax.shift_right_logical(u, 7)
        ibb_v[pl.ds(j * L, L)] = lax.shift_right_logical(i, 7)
        return 0
    lax.fori_loop(0, BPW // L, prep, 0)
    tcp.wait()

    jv = lax.iota(jnp.int32, L)

    def chunk(c, _):
        cb = c * CH
        c1 = pltpu.async_copy(utab.at[uidx_v.at[pl.ds(cb, L)]], ubuf, sem)
        c2 = pltpu.async_copy(itab.at[iidx_v.at[pl.ds(cb, L)]], ibuf, sem)
        c3 = pltpu.async_copy(ubp.at[ubb_v.at[pl.ds(cb, L)]], ubbuf, sem)
        c4 = pltpu.async_copy(ibp.at[ibb_v.at[pl.ds(cb, L)]], ibbuf, sem)
        c1.wait(); c2.wait(); c3.wait(); c4.wait()

        uvec = uidx_v[pl.ds(cb, L)]
        ivec = iidx_v[pl.ds(cb, L)]
        acc = (plsc.load_gather(ubbuf, [jv, lax.bitwise_and(uvec, 127)])
               + plsc.load_gather(ibbuf, [jv, lax.bitwise_and(ivec, 127)]))
        for f in range(F):
            fv = jnp.full((L,), f, jnp.int32)
            u = plsc.load_gather(ubuf, [jv, fv])
            i = plsc.load_gather(ibuf, [jv, fv])
            t = tbufT[f, pl.ds(cb, L)]
            acc = acc + u * (i + t)
        out_v[pl.ds(cb, L)] = acc
        return 0

    lax.fori_loop(0, NCH, chunk, 0)
    pltpu.sync_copy(out_v, out2.at[0, pl.ds(base, BPW)])


def kernel(user_id, item_id, text_embeddings, user_table, item_table,
           user_bias, item_bias):
    nu = user_table.shape[0]
    ni = item_table.shape[0]
    nup = (nu + 127) // 128 * 128
    nip = (ni + 127) // 128 * 128
    ubp = jnp.pad(user_bias, ((0, nup - nu), (0, 0))).reshape(nup // 128, 128)
    ibp = jnp.pad(item_bias, ((0, nip - ni), (0, 0))).reshape(nip // 128, 128)

    mesh = plsc.VectorSubcoreMesh(core_axis_name="c", subcore_axis_name="s")
    k = functools.partial(
        pl.kernel,
        out_type=jax.ShapeDtypeStruct((1, B), jnp.float32),
        mesh=mesh,
        compiler_params=pltpu.CompilerParams(
            needs_layout_passes=False, use_tc_tiling_on_sc=False),
        scratch_types=[
            pltpu.VMEM((BPW,), jnp.int32),
            pltpu.VMEM((BPW,), jnp.int32),
            pltpu.VMEM((BPW,), jnp.int32),
            pltpu.VMEM((BPW,), jnp.int32),
            pltpu.VMEM((CH, F), jnp.float32),
            pltpu.VMEM((CH, F), jnp.float32),
            pltpu.VMEM((F, BPW), jnp.float32),
            pltpu.VMEM((CH, 128), jnp.float32),
            pltpu.VMEM((CH, 128), jnp.float32),
            pltpu.VMEM((BPW,), jnp.float32),
            pltpu.SemaphoreType.DMA,
        ],
    )(_body)
    out2 = k(user_id.T, item_id.T, text_embeddings.T,
             user_table, item_table, ubp, ibp)
    return out2.T


# speedup vs baseline: 1.1006x; 1.0028x over previous
import functools

import jax
import jax.numpy as jnp
from jax import lax
from jax.experimental import pallas as pl
from jax.experimental.pallas import tpu as pltpu
from jax.experimental.pallas import tpu_sc as plsc

B = 16384
F = 64
NC = 2
NS = 16
NW = NC * NS
BPW = B // NW
L = 16
CH = 16
NCH = BPW // CH


def _body(uidT, iidT, textT, utab, itab, ubp, ibp, out2,
          uidx_v, iidx_v, ubb_v, ibb_v, ubuf, ibuf, tbufT, ubbuf, ibbuf,
          out_v, sem):
    wid = lax.axis_index("s") * NC + lax.axis_index("c")
    base = wid * BPW

    pltpu.sync_copy(uidT.at[0, pl.ds(base, BPW)], uidx_v)
    pltpu.sync_copy(iidT.at[0, pl.ds(base, BPW)], iidx_v)
    tcp = pltpu.async_copy(textT.at[:, pl.ds(base, BPW)], tbufT, sem)

    def prep(j, _):
        u = uidx_v[pl.ds(j * L, L)]
        i = iidx_v[pl.ds(j * L, L)]
        ubb_v[pl.ds(j * L, L)] = lax.shift_right_logical(u, 7)
        ibb_v[pl.ds(j * L, L)] = lax.shift_right_logical(i, 7)
        return 0
    lax.fori_loop(0, BPW // L, prep, 0)
    tcp.wait()

    jv = lax.iota(jnp.int32, L)

    def chunk(c, _):
        cb = c * CH
        c1 = pltpu.async_copy(utab.at[uidx_v.at[pl.ds(cb, L)]], ubuf, sem)
        c2 = pltpu.async_copy(itab.at[iidx_v.at[pl.ds(cb, L)]], ibuf, sem)
        c3 = pltpu.async_copy(ubp.at[ubb_v.at[pl.ds(cb, L)]], ubbuf, sem)
        c4 = pltpu.async_copy(ibp.at[ibb_v.at[pl.ds(cb, L)]], ibbuf, sem)
        c1.wait(); c2.wait(); c3.wait(); c4.wait()

        uvec = uidx_v[pl.ds(cb, L)]
        ivec = iidx_v[pl.ds(cb, L)]
        acc = (plsc.load_gather(ubbuf, [jv, lax.bitwise_and(uvec, 127)])
               + plsc.load_gather(ibbuf, [jv, lax.bitwise_and(ivec, 127)]))
        for f in range(F):
            fv = jnp.full((L,), f, jnp.int32)
            u = plsc.load_gather(ubuf, [jv, fv])
            i = plsc.load_gather(ibuf, [jv, fv])
            t = tbufT[f, pl.ds(cb, L)]
            acc = acc + u * (i + t)
        out_v[pl.ds(cb, L)] = acc
        return 0

    lax.fori_loop(0, NCH, chunk, 0)
    pltpu.sync_copy(out_v, out2.at[0, pl.ds(base, BPW)])


def kernel(user_id, item_id, text_embeddings, user_table, item_table,
           user_bias, item_bias):
    nu = user_table.shape[0]
    ni = item_table.shape[0]
    nup = (nu + 127) // 128 * 128
    nip = (ni + 127) // 128 * 128
    ubp = jnp.pad(user_bias.T, ((0, 0), (0, nup - nu))).reshape(nup // 128, 128)
    ibp = jnp.pad(item_bias.T, ((0, 0), (0, nip - ni))).reshape(nip // 128, 128)

    mesh = plsc.VectorSubcoreMesh(core_axis_name="c", subcore_axis_name="s")
    k = functools.partial(
        pl.kernel,
        out_type=jax.ShapeDtypeStruct((1, B), jnp.float32),
        mesh=mesh,
        compiler_params=pltpu.CompilerParams(
            needs_layout_passes=False, use_tc_tiling_on_sc=False),
        scratch_types=[
            pltpu.VMEM((BPW,), jnp.int32),
            pltpu.VMEM((BPW,), jnp.int32),
            pltpu.VMEM((BPW,), jnp.int32),
            pltpu.VMEM((BPW,), jnp.int32),
            pltpu.VMEM((CH, F), jnp.float32),
            pltpu.VMEM((CH, F), jnp.float32),
            pltpu.VMEM((F, BPW), jnp.float32),
            pltpu.VMEM((CH, 128), jnp.float32),
            pltpu.VMEM((CH, 128), jnp.float32),
            pltpu.VMEM((BPW,), jnp.float32),
            pltpu.SemaphoreType.DMA,
        ],
    )(_body)
    out2 = k(user_id.T, item_id.T, text_embeddings.T,
             user_table, item_table, ubp, ibp)
    return out2.T


# speedup vs baseline: 1.1035x; 1.0027x over previous
import functools

import jax
import jax.numpy as jnp
from jax import lax
from jax.experimental import pallas as pl
from jax.experimental.pallas import tpu as pltpu
from jax.experimental.pallas import tpu_sc as plsc

B = 16384
F = 64
NC = 2
NS = 16
NW = NC * NS
BPW = B // NW
L = 16
CH = 16
NCH = BPW // CH


def _body(uidT, iidT, textT, utab, itab, ubT, ibT, out2,
          uidx_v, iidx_v, ubuf, ibuf, tbufT, ubbuf, ibbuf,
          out_v, sem):
    wid = lax.axis_index("s") * NC + lax.axis_index("c")
    base = wid * BPW

    pltpu.sync_copy(uidT.at[0, pl.ds(base, BPW)], uidx_v)
    pltpu.sync_copy(iidT.at[0, pl.ds(base, BPW)], iidx_v)
    tcp = pltpu.async_copy(textT.at[:, pl.ds(base, BPW)], tbufT, sem)

    tcp.wait()

    jv = lax.iota(jnp.int32, L)

    def chunk(c, _):
        cb = c * CH
        uvec = uidx_v[pl.ds(cb, L)]
        ivec = iidx_v[pl.ds(cb, L)]
        usubv = lax.bitwise_and(uvec, 7)
        isubv = lax.bitwise_and(ivec, 7)
        ubasev = uvec - usubv
        ibasev = ivec - isubv
        copies = [
            pltpu.async_copy(utab.at[uidx_v.at[pl.ds(cb, L)]], ubuf, sem),
            pltpu.async_copy(itab.at[iidx_v.at[pl.ds(cb, L)]], ibuf, sem),
        ]
        for j in range(CH):
            ub_ = pl.multiple_of(ubasev[j], 8)
            ib_ = pl.multiple_of(ibasev[j], 8)
            copies.append(pltpu.async_copy(
                ubT.at[0, pl.ds(ub_, 8)], ubbuf.at[pl.ds(j * 8, 8)], sem))
            copies.append(pltpu.async_copy(
                ibT.at[0, pl.ds(ib_, 8)], ibbuf.at[pl.ds(j * 8, 8)], sem))
        for cp in copies:
            cp.wait()

        acc = (plsc.load_gather(ubbuf, [jv * 8 + usubv])
               + plsc.load_gather(ibbuf, [jv * 8 + isubv]))
        for f in range(F):
            fv = jnp.full((L,), f, jnp.int32)
            u = plsc.load_gather(ubuf, [jv, fv])
            i = plsc.load_gather(ibuf, [jv, fv])
            t = tbufT[f, pl.ds(cb, L)]
            acc = acc + u * (i + t)
        out_v[pl.ds(cb, L)] = acc
        return 0

    lax.fori_loop(0, NCH, chunk, 0)
    pltpu.sync_copy(out_v, out2.at[0, pl.ds(base, BPW)])


def kernel(user_id, item_id, text_embeddings, user_table, item_table,
           user_bias, item_bias):
    mesh = plsc.VectorSubcoreMesh(core_axis_name="c", subcore_axis_name="s")
    k = functools.partial(
        pl.kernel,
        out_type=jax.ShapeDtypeStruct((1, B), jnp.float32),
        mesh=mesh,
        compiler_params=pltpu.CompilerParams(
            needs_layout_passes=False, use_tc_tiling_on_sc=False),
        scratch_types=[
            pltpu.VMEM((BPW,), jnp.int32),
            pltpu.VMEM((BPW,), jnp.int32),
            pltpu.VMEM((CH, F), jnp.float32),
            pltpu.VMEM((CH, F), jnp.float32),
            pltpu.VMEM((F, BPW), jnp.float32),
            pltpu.VMEM((CH * 8,), jnp.float32),
            pltpu.VMEM((CH * 8,), jnp.float32),
            pltpu.VMEM((BPW,), jnp.float32),
            pltpu.SemaphoreType.DMA,
        ],
    )(_body)
    out2 = k(user_id.T, item_id.T, text_embeddings.T,
             user_table, item_table, user_bias.T, item_bias.T)
    return out2.T


# speedup vs baseline: 1.1208x; 1.0157x over previous
import functools

import jax
import jax.numpy as jnp
from jax import lax
from jax.experimental import pallas as pl
from jax.experimental.pallas import tpu as pltpu
from jax.experimental.pallas import tpu_sc as plsc

B = 16384
F = 64
NC = 2
NS = 16
NW = NC * NS
BPW = B // NW
L = 16


def _body(uid, iid, text, utab, itab, ubias, ibias, out,
          uidx_v, iidx_v, urows, irows, trows, ub_v, ib_v, out_v, sem):
    wid = lax.axis_index("s") * NC + lax.axis_index("c")
    base = wid * BPW

    pltpu.sync_copy(uid.at[pl.ds(base, BPW)], uidx_v)
    pltpu.sync_copy(iid.at[pl.ds(base, BPW)], iidx_v)

    c1 = pltpu.async_copy(utab.at[uidx_v], urows, sem)
    c2 = pltpu.async_copy(itab.at[iidx_v], irows, sem)
    c3 = pltpu.async_copy(ubias.at[uidx_v], ub_v, sem)
    c4 = pltpu.async_copy(ibias.at[iidx_v], ib_v, sem)
    c5 = pltpu.async_copy(text.at[pl.ds(base, BPW)], trows, sem)
    c1.wait(); c2.wait(); c3.wait(); c4.wait(); c5.wait()

    def group(g, _):
        rb = g * L
        ridx = rb + lax.iota(jnp.int32, L)
        acc = ub_v[pl.ds(rb, L)] + ib_v[pl.ds(rb, L)]

        def col(f, acc):
            cidx = jnp.full((L,), f, jnp.int32)
            u = plsc.load_gather(urows, [ridx, cidx])
            i = plsc.load_gather(irows, [ridx, cidx])
            t = plsc.load_gather(trows, [ridx, cidx])
            return acc + u * (i + t)

        acc = lax.fori_loop(0, F, col, acc, unroll=8)
        out_v[pl.ds(rb, L)] = acc
        return 0

    lax.fori_loop(0, BPW // L, group, 0)
    pltpu.sync_copy(out_v, out.at[pl.ds(base, BPW)])


def kernel(user_id, item_id, text_embeddings, user_table, item_table,
           user_bias, item_bias):
    mesh = plsc.VectorSubcoreMesh(core_axis_name="c", subcore_axis_name="s")
    k = functools.partial(
        pl.kernel,
        out_type=jax.ShapeDtypeStruct((B,), jnp.float32),
        mesh=mesh,
        compiler_params=pltpu.CompilerParams(
            needs_layout_passes=False, use_tc_tiling_on_sc=False),
        scratch_types=[
            pltpu.VMEM((BPW,), jnp.int32),
            pltpu.VMEM((BPW,), jnp.int32),
            pltpu.VMEM((BPW, F), jnp.float32),
            pltpu.VMEM((BPW, F), jnp.float32),
            pltpu.VMEM((BPW, F), jnp.float32),
            pltpu.VMEM((BPW,), jnp.float32),
            pltpu.VMEM((BPW,), jnp.float32),
            pltpu.VMEM((BPW,), jnp.float32),
            pltpu.SemaphoreType.DMA,
        ],
    )(_body)
    out = k(user_id.reshape(B), item_id.reshape(B), text_embeddings,
            user_table, item_table,
            user_bias.reshape(user_bias.shape[0]),
            item_bias.reshape(item_bias.shape[0]))
    return out.reshape(B, 1)
